# Initial kernel scaffold; baseline (speedup 1.0000x reference)
import jax
import jax.numpy as jnp
from jax.experimental import pallas as pl


def kernel(x, card_buffer):
    raise NotImplementedError("write your pallas kernel here")


# speedup vs baseline: 11.0284x; 11.0284x over previous
import functools

import jax
import jax.numpy as jnp
from jax.experimental import pallas as pl
from jax.experimental.pallas import tpu as pltpu

_B, _I, _E = 16384, 128, 18
_LO, _HI = 64, 71
_NC = _HI - _LO
_W = _I * _E
_CLO = _LO * _E
_CHI = _HI * _E
_TB = 512


def _body(x_ref, m_ref, r2_ref, bd_ref, o_ref):
    x = x_ref[...]
    o_ref[...] = jnp.dot(x, m_ref[...], preferred_element_type=jnp.float32)
    xs_rep = jnp.dot(x, r2_ref[...], preferred_element_type=jnp.float32)
    mi = jax.lax.broadcasted_iota(jnp.int32, xs_rep.shape, 1)
    oh = (xs_rep == (mi % 52).astype(jnp.float32)).astype(jnp.float32)
    o_ref[:, _CLO:_CHI] = jnp.dot(oh, bd_ref[...], preferred_element_type=jnp.float32)


@jax.jit
def kernel(x, card_buffer):
    if x.ndim == 3:
        x = x[:, 0, :]
    B = x.shape[0]
    f32 = jnp.float32
    M = (jnp.arange(_W)[None, :] // _E == jnp.arange(_I)[:, None]).astype(f32)
    R2 = (jnp.arange(_I)[:, None] == _LO + jnp.arange(_NC * 52)[None, :] // 52).astype(f32)
    BD = (jnp.eye(_NC, dtype=f32)[:, None, :, None]
          * card_buffer[None, :, None, :]).reshape(_NC * 52, _NC * _E)

    out2d = pl.pallas_call(
        _body,
        grid=(B // _TB,),
        in_specs=[
            pl.BlockSpec((_TB, _I), lambda i: (i, 0)),
            pl.BlockSpec((_I, _W), lambda i: (0, 0)),
            pl.BlockSpec((_I, _NC * 52), lambda i: (0, 0)),
            pl.BlockSpec((_NC * 52, _NC * _E), lambda i: (0, 0)),
        ],
        out_specs=pl.BlockSpec((_TB, _W), lambda i: (i, 0)),
        out_shape=jax.ShapeDtypeStruct((B, _W), f32),
        compiler_params=pltpu.CompilerParams(
            dimension_semantics=("parallel",),
        ),
    )(x, M, R2, BD)
    return out2d.reshape(B, _I, _E)
